# Initial kernel scaffold; baseline (speedup 1.0000x reference)
import jax
import jax.numpy as jnp
from jax.experimental import pallas as pl


def kernel(input, W_subject, W_event, W_objectType, W_objectData, W_red, b_red):
    raise NotImplementedError("write your pallas kernel here")


# speedup vs baseline: 6.3513x; 6.3513x over previous
import functools

import jax
import jax.numpy as jnp
from jax import lax
from jax.experimental import pallas as pl
from jax.experimental.pallas import tpu as pltpu
from jax.experimental.pallas import tpu_sc as plsc

B, L = 4096, 50
V = 100000
D0, D1, D2 = 32, 16, 16
DP, D3 = 128, 32
DOUT = D0 + D1 + D2 + D3
N = B * L

NC, NS = 2, 16
NW = NC * NS
PER_W = N // NW
CHUNK = 128
NCHUNK = PER_W // CHUNK


def _fold_body(wd_ref, wr_ref, br_ref, out_ref):
    out_ref[...] = (
        jnp.dot(wd_ref[...], wr_ref[...], preferred_element_type=jnp.float32)
        + br_ref[...]
    )


def _fold_table(W_objectData, W_red, b_red):
    rows = 2000
    grid = (V // rows,)
    return pl.pallas_call(
        _fold_body,
        grid=grid,
        in_specs=[
            pl.BlockSpec((rows, DP), lambda i: (i, 0)),
            pl.BlockSpec((DP, D3), lambda i: (0, 0)),
            pl.BlockSpec((1, D3), lambda i: (0, 0)),
        ],
        out_specs=pl.BlockSpec((rows, D3), lambda i: (i, 0)),
        out_shape=jax.ShapeDtypeStruct((V, D3), jnp.float32),
    )(W_objectData, W_red, b_red.reshape(1, D3))


def _sc_body(idx_hbm, t0, t1, t2, t3, out_hbm,
             idx_v, b0, b1, b2, b3, sem):
    wid = lax.axis_index("s") * NC + lax.axis_index("c")
    base = wid * PER_W
    pltpu.sync_copy(idx_hbm.at[wid], idx_v)

    def body(c, _):
        d0 = pltpu.async_copy(t0.at[idx_v.at[c, 0]], b0, sem)
        d1 = pltpu.async_copy(t1.at[idx_v.at[c, 1]], b1, sem)
        d2 = pltpu.async_copy(t2.at[idx_v.at[c, 2]], b2, sem)
        d3 = pltpu.async_copy(t3.at[idx_v.at[c, 3]], b3, sem)
        d0.wait()
        d1.wait()
        d2.wait()
        d3.wait()
        rb = base + c * CHUNK
        pltpu.sync_copy(b0, out_hbm.at[pl.ds(rb, CHUNK), pl.ds(0, D0)])
        pltpu.sync_copy(b1, out_hbm.at[pl.ds(rb, CHUNK), pl.ds(D0, D1)])
        pltpu.sync_copy(b2, out_hbm.at[pl.ds(rb, CHUNK), pl.ds(D0 + D1, D2)])
        pltpu.sync_copy(b3, out_hbm.at[pl.ds(rb, CHUNK), pl.ds(D0 + D1 + D2, D3)])
        return ()

    lax.fori_loop(0, NCHUNK, body, ())


_sc_gather = functools.partial(
    pl.kernel,
    out_type=jax.ShapeDtypeStruct((N, DOUT), jnp.float32),
    mesh=plsc.VectorSubcoreMesh(core_axis_name="c", subcore_axis_name="s"),
    compiler_params=pltpu.CompilerParams(use_tc_tiling_on_sc=False),
    scratch_types=[
        pltpu.VMEM((NCHUNK, 4, CHUNK), jnp.int32),
        pltpu.VMEM((CHUNK, D0), jnp.float32),
        pltpu.VMEM((CHUNK, D1), jnp.float32),
        pltpu.VMEM((CHUNK, D2), jnp.float32),
        pltpu.VMEM((CHUNK, D3), jnp.float32),
        pltpu.SemaphoreType.DMA,
    ],
)(_sc_body)


def kernel(input, W_subject, W_event, W_objectType, W_objectData, W_red, b_red):
    folded = _fold_table(W_objectData, W_red, b_red)
    idx = input.reshape(NW, NCHUNK, CHUNK, 4).transpose(0, 1, 3, 2)
    out = _sc_gather(idx, W_subject, W_event, W_objectType, folded)
    return out.reshape(B, L, DOUT)


# speedup vs baseline: 6.8599x; 1.0801x over previous
import functools

import jax
import jax.numpy as jnp
from jax import lax
from jax.experimental import pallas as pl
from jax.experimental.pallas import tpu as pltpu
from jax.experimental.pallas import tpu_sc as plsc

B, L = 4096, 50
V = 100000
D0, D1, D2 = 32, 16, 16
DP, D3 = 128, 32
DOUT = D0 + D1 + D2 + D3
N = B * L

NC, NS = 2, 16
NW = NC * NS
PER_W = N // NW
CHUNK = 128
NCHUNK = PER_W // CHUNK


def _fold_body(wd_ref, wr_ref, br_ref, out_ref):
    out_ref[...] = (
        jnp.dot(wd_ref[...], wr_ref[...], preferred_element_type=jnp.float32)
        + br_ref[...]
    )


def _fold_table(W_objectData, W_red, b_red):
    rows = 2000
    grid = (V // rows,)
    return pl.pallas_call(
        _fold_body,
        grid=grid,
        in_specs=[
            pl.BlockSpec((rows, DP), lambda i: (i, 0)),
            pl.BlockSpec((DP, D3), lambda i: (0, 0)),
            pl.BlockSpec((1, D3), lambda i: (0, 0)),
        ],
        out_specs=pl.BlockSpec((rows, D3), lambda i: (i, 0)),
        out_shape=jax.ShapeDtypeStruct((V, D3), jnp.float32),
    )(W_objectData, W_red, b_red.reshape(1, D3))


NBUF = 2
NSTEP = NCHUNK // NBUF

_OFFS = (0, D0, D0 + D1, D0 + D1 + D2)
_DIMS = (D0, D1, D2, D3)


def _sc_body(idx_hbm, t0, t1, t2, t3, out_hbm,
             idx_v, bufs, gsems, wsems):
    wid = lax.axis_index("s") * NC + lax.axis_index("c")
    base = wid * PER_W
    tabs = (t0, t1, t2, t3)
    pltpu.sync_copy(idx_hbm.at[wid], idx_v)

    def fire_gathers(b, c):
        for f in range(4):
            pltpu.make_async_copy(
                tabs[f].at[idx_v.at[c, f]], bufs[b][f], gsems[b]).start()

    def wait_gathers(b, c):
        for f in range(4):
            pltpu.make_async_copy(
                tabs[f].at[idx_v.at[c, f]], bufs[b][f], gsems[b]).wait()

    class _W:
        def __init__(self, b, c):
            rb = base + c * CHUNK
            self.ds = [
                pltpu.make_async_copy(
                    bufs[b][f],
                    out_hbm.at[pl.ds(rb, CHUNK), pl.ds(_OFFS[f], _DIMS[f])],
                    wsems[b])
                for f in range(4)
            ]

        def start(self):
            for d in self.ds:
                d.start()

        def wait(self):
            for d in self.ds:
                d.wait()

    write = _W

    for b in range(NBUF):
        fire_gathers(b, b)

    def body(s, _):
        c0 = s * NBUF
        for b in range(NBUF):
            wait_gathers(b, c0 + b)
            write(b, c0 + b).start()
        for b in range(NBUF):
            write(b, c0 + b).wait()
            fire_gathers(b, c0 + NBUF + b)
        return ()

    lax.fori_loop(0, NSTEP - 1, body, ())

    c0 = (NSTEP - 1) * NBUF
    for b in range(NBUF):
        wait_gathers(b, c0 + b)
        write(b, c0 + b).start()
    for b in range(NBUF):
        write(b, c0 + b).wait()


_sc_gather = functools.partial(
    pl.kernel,
    out_type=jax.ShapeDtypeStruct((N, DOUT), jnp.float32),
    mesh=plsc.VectorSubcoreMesh(core_axis_name="c", subcore_axis_name="s"),
    compiler_params=pltpu.CompilerParams(use_tc_tiling_on_sc=False),
    scratch_types=[
        pltpu.VMEM((NCHUNK, 4, CHUNK), jnp.int32),
        [[pltpu.VMEM((CHUNK, d), jnp.float32) for d in _DIMS]
         for _ in range(NBUF)],
        [pltpu.SemaphoreType.DMA for _ in range(NBUF)],
        [pltpu.SemaphoreType.DMA for _ in range(NBUF)],
    ],
)(_sc_body)


def kernel(input, W_subject, W_event, W_objectType, W_objectData, W_red, b_red):
    folded = _fold_table(W_objectData, W_red, b_red)
    idx = input.reshape(NW, NCHUNK, CHUNK, 4).transpose(0, 1, 3, 2)
    out = _sc_gather(idx, W_subject, W_event, W_objectType, folded)
    return out.reshape(B, L, DOUT)
